# Initial kernel scaffold; baseline (speedup 1.0000x reference)
import functools

import jax
import jax.numpy as jnp
from jax import lax
from jax.experimental import pallas as pl
from jax.experimental.pallas import tpu as pltpu
from jax.experimental.pallas import tpu_sc as plsc

BATCH = 4096
SEQ = 50
EMB = 128
TOTAL = BATCH * SEQ

_info = plsc.get_sparse_core_info()
NC = _info.num_cores
NS = _info.num_subcores
NW = NC * NS
PER_W = TOTAL // NW
CHUNK = 128
NCH = PER_W // CHUNK
IDX_ROWS = TOTAL // CHUNK

_mesh = plsc.VectorSubcoreMesh(core_axis_name="c", subcore_axis_name="s")


@functools.partial(
    pl.kernel,
    mesh=_mesh,
    out_type=jax.ShapeDtypeStruct((TOTAL, EMB), jnp.float32),
    scratch_types=[
        pltpu.VMEM((NCH, CHUNK), jnp.int32),
        pltpu.VMEM((CHUNK, EMB), jnp.float32),
        pltpu.SemaphoreType.DMA,
    ],
)
def _emb_lookup(idx_hbm, table_hbm, out_hbm, idx_v, rows_v, sem):
    wid = lax.axis_index("s") * NC + lax.axis_index("c")
    row0 = wid * NCH
    base0 = wid * PER_W
    pltpu.sync_copy(idx_hbm.at[pl.ds(row0, NCH), :], idx_v)

    def body(j, carry):
        pltpu.async_copy(table_hbm.at[idx_v.at[j]], rows_v, sem).wait()
        pltpu.sync_copy(rows_v, out_hbm.at[pl.ds(base0 + j * CHUNK, CHUNK), :])
        return carry

    lax.fori_loop(0, NCH, body, 0)


def kernel(input_tensor, weight):
    idx = input_tensor.reshape(IDX_ROWS, CHUNK).astype(jnp.int32)
    out = _emb_lookup(idx, weight)
    return out.reshape(BATCH, SEQ, EMB)


# speedup vs baseline: 2.9605x; 2.9605x over previous
import functools

import jax
import jax.numpy as jnp
from jax import lax
from jax.experimental import pallas as pl
from jax.experimental.pallas import tpu as pltpu
from jax.experimental.pallas import tpu_sc as plsc

BATCH = 4096
SEQ = 50
EMB = 128
TOTAL = BATCH * SEQ

_info = plsc.get_sparse_core_info()
NC = _info.num_cores
NS = _info.num_subcores
NW = NC * NS
PER_W = TOTAL // NW
CHUNK = 128
NCH = PER_W // CHUNK
IDX_ROWS = TOTAL // CHUNK

_mesh = plsc.VectorSubcoreMesh(core_axis_name="c", subcore_axis_name="s")


@functools.partial(
    pl.kernel,
    mesh=_mesh,
    out_type=jax.ShapeDtypeStruct((TOTAL, EMB), jnp.float32),
    scratch_types=[
        pltpu.VMEM((NCH, CHUNK), jnp.int32),
        pltpu.VMEM((CHUNK, EMB), jnp.float32),
        pltpu.SemaphoreType.DMA,
    ],
)
def _emb_lookup(idx_hbm, table_hbm, out_hbm, idx_v, rows_v, sem):
    wid = lax.axis_index("s") * NC + lax.axis_index("c")
    base0 = wid * PER_W
    pltpu.sync_copy(idx_hbm.at[wid], idx_v)

    def body(j, carry):
        pltpu.async_copy(table_hbm.at[idx_v.at[j]], rows_v, sem).wait()
        pltpu.sync_copy(rows_v, out_hbm.at[pl.ds(base0 + j * CHUNK, CHUNK), :])
        return carry

    lax.fori_loop(0, NCH, body, 0)


def kernel(input_tensor, weight):
    idx = input_tensor.reshape(NW, NCH, CHUNK).astype(jnp.int32)
    out = _emb_lookup(idx, weight)
    return out.reshape(BATCH, SEQ, EMB)


# speedup vs baseline: 3.3428x; 1.1291x over previous
import functools

import jax
import jax.numpy as jnp
from jax import lax
from jax.experimental import pallas as pl
from jax.experimental.pallas import tpu as pltpu
from jax.experimental.pallas import tpu_sc as plsc

BATCH = 4096
SEQ = 50
EMB = 128
TOTAL = BATCH * SEQ

_info = plsc.get_sparse_core_info()
NC = _info.num_cores
NS = _info.num_subcores
NW = NC * NS
PER_W = TOTAL // NW
CHUNK = 128
NCH = PER_W // CHUNK
IDX_ROWS = TOTAL // CHUNK

_mesh = plsc.VectorSubcoreMesh(core_axis_name="c", subcore_axis_name="s")


NBUF = 5
NOUT = NCH // NBUF - 1


@functools.partial(
    pl.kernel,
    mesh=_mesh,
    out_type=jax.ShapeDtypeStruct((TOTAL, EMB), jnp.float32),
    scratch_types=[
        pltpu.VMEM((NCH, CHUNK), jnp.int32),
        pltpu.VMEM((NBUF * CHUNK, EMB), jnp.float32),
    ]
    + [pltpu.SemaphoreType.DMA] * (2 * NBUF),
)
def _emb_lookup(idx_hbm, table_hbm, out_hbm, idx_v, rows_v, *sems):
    gsem = sems[:NBUF]
    wsem = sems[NBUF:]
    wid = lax.axis_index("s") * NC + lax.axis_index("c")
    base0 = wid * PER_W
    pltpu.sync_copy(idx_hbm.at[wid], idx_v)

    def buf(b):
        return rows_v.at[pl.ds(b * CHUNK, CHUNK), :]

    def out_slot(j):
        return out_hbm.at[pl.ds(base0 + j * CHUNK, CHUNK), :]

    def gather_start(j, b):
        pltpu.async_copy(table_hbm.at[idx_v.at[j]], buf(b), gsem[b])

    def gather_wait(j, b):
        pltpu.make_async_copy(table_hbm.at[idx_v.at[j]], buf(b), gsem[b]).wait()

    def wb_start(j, b):
        pltpu.async_copy(buf(b), out_slot(j), wsem[b])

    def wb_wait(j, b):
        pltpu.make_async_copy(buf(b), out_slot(j), wsem[b]).wait()

    for b in range(NBUF):
        gather_start(b, b)

    def body(g, carry):
        j0 = g * NBUF
        for b in range(NBUF):
            gather_wait(j0 + b, b)
            wb_start(j0 + b, b)
            wb_wait(j0 + b, b)
            gather_start(j0 + NBUF + b, b)
        return carry

    lax.fori_loop(0, NOUT, body, 0)

    j0 = NOUT * NBUF
    for b in range(NBUF):
        gather_wait(j0 + b, b)
        wb_start(j0 + b, b)
    for b in range(NBUF):
        wb_wait(j0 + b, b)


def kernel(input_tensor, weight):
    idx = input_tensor.reshape(NW, NCH, CHUNK).astype(jnp.int32)
    out = _emb_lookup(idx, weight)
    return out.reshape(BATCH, SEQ, EMB)


# speedup vs baseline: 3.3555x; 1.0038x over previous
import functools

import jax
import jax.numpy as jnp
from jax import lax
from jax.experimental import pallas as pl
from jax.experimental.pallas import tpu as pltpu
from jax.experimental.pallas import tpu_sc as plsc

BATCH = 4096
SEQ = 50
EMB = 128
TOTAL = BATCH * SEQ

_info = plsc.get_sparse_core_info()
NC = _info.num_cores
NS = _info.num_subcores
NW = NC * NS
PER_W = TOTAL // NW
CHUNK = 128
NCH = PER_W // CHUNK
IDX_ROWS = TOTAL // CHUNK

_mesh = plsc.VectorSubcoreMesh(core_axis_name="c", subcore_axis_name="s")


NBUF = 5
G = 3
W = NBUF - G
NLAP = NCH // NBUF


@functools.partial(
    pl.kernel,
    mesh=_mesh,
    out_type=jax.ShapeDtypeStruct((TOTAL, EMB), jnp.float32),
    scratch_types=[
        pltpu.VMEM((NCH, CHUNK), jnp.int32),
        pltpu.VMEM((NBUF * CHUNK, EMB), jnp.float32),
    ]
    + [pltpu.SemaphoreType.DMA] * (2 * NBUF),
)
def _emb_lookup(idx_hbm, table_hbm, out_hbm, idx_v, rows_v, *sems):
    gsem = sems[:NBUF]
    wsem = sems[NBUF:]
    wid = lax.axis_index("s") * NC + lax.axis_index("c")
    base0 = wid * PER_W
    pltpu.sync_copy(idx_hbm.at[wid], idx_v)

    def buf(b):
        return rows_v.at[pl.ds(b * CHUNK, CHUNK), :]

    def out_slot(j):
        return out_hbm.at[pl.ds(base0 + j * CHUNK, CHUNK), :]

    def gather_start(j, b):
        pltpu.async_copy(table_hbm.at[idx_v.at[j]], buf(b), gsem[b])

    def gather_wait(j, b):
        pltpu.make_async_copy(table_hbm.at[idx_v.at[j]], buf(b), gsem[b]).wait()

    def wb_start(j, b):
        pltpu.async_copy(buf(b), out_slot(j), wsem[b])

    def wb_wait(j, b):
        pltpu.make_async_copy(buf(b), out_slot(j), wsem[b]).wait()

    def step(j, b, do_wb_wait, do_gather):
        gather_wait(j, b)
        wb_start(j, b)
        if do_wb_wait:
            wb_wait(j - W, (b - W) % NBUF)
        if do_gather:
            gather_start(j + G, (b + G) % NBUF)

    for b in range(G):
        gather_start(b, b)

    for b in range(NBUF):
        step(b, b, b >= W, True)

    def body(g, carry):
        j0 = g * NBUF
        for b in range(NBUF):
            step(j0 + b, b, True, True)
        return carry

    lax.fori_loop(1, NLAP - 1, body, 0)

    j0 = (NLAP - 1) * NBUF
    for b in range(NBUF):
        step(j0 + b, b, True, j0 + b + G < NCH)

    for b in range(W):
        j = NCH - W + b
        wb_wait(j, j % NBUF)


def kernel(input_tensor, weight):
    idx = input_tensor.reshape(NW, NCH, CHUNK).astype(jnp.int32)
    out = _emb_lookup(idx, weight)
    return out.reshape(BATCH, SEQ, EMB)


# speedup vs baseline: 5.9634x; 1.7772x over previous
import functools

import jax
import jax.numpy as jnp
from jax import lax
from jax.experimental import pallas as pl
from jax.experimental.pallas import tpu as pltpu
from jax.experimental.pallas import tpu_sc as plsc

BATCH = 4096
SEQ = 50
EMB = 128

_info = plsc.get_sparse_core_info()
NC = _info.num_cores
NS = _info.num_subcores
NW = NC * NS
PER_W = BATCH // NW

NBUF = 8
G = 5
W = NBUF - G
NLAP = PER_W // NBUF

_mesh = plsc.VectorSubcoreMesh(core_axis_name="c", subcore_axis_name="s")


@functools.partial(
    pl.kernel,
    mesh=_mesh,
    out_type=jax.ShapeDtypeStruct((BATCH, SEQ, EMB), jnp.float32),
    scratch_types=[
        pltpu.VMEM((PER_W, SEQ), jnp.int32),
        pltpu.VMEM((NBUF, SEQ, EMB), jnp.float32),
    ]
    + [pltpu.SemaphoreType.DMA] * (2 * NBUF),
)
def _emb_lookup(idx_hbm, table_hbm, out_hbm, idx_v, rows_v, *sems):
    gsem = sems[:NBUF]
    wsem = sems[NBUF:]
    wid = lax.axis_index("s") * NC + lax.axis_index("c")
    base0 = wid * PER_W
    pltpu.sync_copy(idx_hbm.at[pl.ds(base0, PER_W), :], idx_v)

    def gather_start(j, b):
        pltpu.async_copy(table_hbm.at[idx_v.at[j]], rows_v.at[b], gsem[b])

    def gather_wait(j, b):
        pltpu.make_async_copy(
            table_hbm.at[idx_v.at[j]], rows_v.at[b], gsem[b]
        ).wait()

    def wb_start(j, b):
        pltpu.async_copy(rows_v.at[b], out_hbm.at[base0 + j], wsem[b])

    def wb_wait(j, b):
        pltpu.make_async_copy(rows_v.at[b], out_hbm.at[base0 + j], wsem[b]).wait()

    def step(j, b, do_wb_wait, do_gather):
        gather_wait(j, b)
        wb_start(j, b)
        if do_wb_wait:
            wb_wait(j - W, (b - W) % NBUF)
        if do_gather:
            gather_start(j + G, (b + G) % NBUF)

    for b in range(G):
        gather_start(b, b)

    for b in range(NBUF):
        step(b, b, b >= W, True)

    def body(g, carry):
        j0 = g * NBUF
        for b in range(NBUF):
            step(j0 + b, b, True, True)
        return carry

    lax.fori_loop(1, NLAP - 1, body, 0)

    j0 = (NLAP - 1) * NBUF
    for b in range(NBUF):
        step(j0 + b, b, True, j0 + b + G < PER_W)

    for b in range(W):
        j = PER_W - W + b
        wb_wait(j, j % NBUF)


def kernel(input_tensor, weight):
    return _emb_lookup(input_tensor.astype(jnp.int32), weight)
